# baseline (speedup 1.0000x reference)
import functools

import jax
import jax.numpy as jnp
from jax import lax
from jax.experimental import pallas as pl
from jax.experimental.pallas import tpu as pltpu
from jax.experimental.pallas import tpu_sc as plsc

BATCH = 16384
EMB = 64
BN_EPS = 1e-5

NC = 2
NS = 16
NW = NC * NS
B_PER_W = BATCH // NW
CHUNK = 64
NCHUNK = B_PER_W // CHUNK


def _sc_gather4(user, item, t_ug, t_ig, t_um, t_im):
    mesh = plsc.VectorSubcoreMesh(core_axis_name="c", subcore_axis_name="s")

    out_t = [jax.ShapeDtypeStruct((BATCH, EMB), jnp.float32)] * 4
    scratch = [
        pltpu.VMEM((B_PER_W,), jnp.int32),
        pltpu.VMEM((B_PER_W,), jnp.int32),
    ] + [pltpu.VMEM((CHUNK, EMB), jnp.float32)] * 4 + [
        pltpu.SemaphoreType.DMA,
    ]

    @functools.partial(pl.kernel, mesh=mesh, out_type=out_t,
                       scratch_types=scratch)
    def k(u_hbm, i_hbm, tug, tig, tum, tim,
          o_ug, o_ig, o_um, o_im,
          idx_u, idx_i, b_ug, b_ig, b_um, b_im, sem):
        wid = lax.axis_index("s") * NC + lax.axis_index("c")
        base = wid * B_PER_W
        pltpu.sync_copy(u_hbm.at[pl.ds(base, B_PER_W)], idx_u)
        pltpu.sync_copy(i_hbm.at[pl.ds(base, B_PER_W)], idx_i)

        tabs = (tug, tig, tum, tim)
        bufs = (b_ug, b_ig, b_um, b_im)
        outs = (o_ug, o_ig, o_um, o_im)

        def chunk_body(c, carry):
            off = c * CHUNK
            cps = []
            for g in range(CHUNK // 16):
                uu = idx_u[pl.ds(off + g * 16, 16)]
                vv = idx_i[pl.ds(off + g * 16, 16)]
                for l in range(16):
                    j = g * 16 + l
                    ru = uu[l]
                    ri = vv[l]
                    cps.append(pltpu.async_copy(
                        tug.at[pl.ds(ru, 1)], b_ug.at[pl.ds(j, 1)], sem))
                    cps.append(pltpu.async_copy(
                        tum.at[pl.ds(ru, 1)], b_um.at[pl.ds(j, 1)], sem))
                    cps.append(pltpu.async_copy(
                        tig.at[pl.ds(ri, 1)], b_ig.at[pl.ds(j, 1)], sem))
                    cps.append(pltpu.async_copy(
                        tim.at[pl.ds(ri, 1)], b_im.at[pl.ds(j, 1)], sem))
            for cp in cps:
                cp.wait()
            for buf, out in zip(bufs, outs):
                pltpu.sync_copy(buf, out.at[pl.ds(base + off, CHUNK)])
            return carry

        lax.fori_loop(0, NCHUNK, chunk_body, 0)

    return k(user, item, t_ug, t_ig, t_um, t_im)


def _tc_dense(ug, ig, um, im, wfg, w0a, w0b, b0r, w1, b1r, w2, b2r,
              w3, b3r, wfm, consts):
    BB = 2048
    grid = BATCH // BB

    def body(ug_r, ig_r, um_r, im_r, wfg_r, w0a_r, w0b_r, b0_r, w1_r, b1_r,
             w2_r, b2_r, w3_r, b3_r, wfm_r, c_r, out_r):
        f32 = jnp.float32
        x = jnp.dot(um_r[...], w0a_r[...], preferred_element_type=f32)
        x = x + jnp.dot(im_r[...], w0b_r[...], preferred_element_type=f32)
        x = jnp.maximum(x + b0_r[...], 0.0)
        x = jnp.maximum(jnp.dot(x, w1_r[...], preferred_element_type=f32)
                        + b1_r[...], 0.0)
        x = jnp.maximum(jnp.dot(x, w2_r[...], preferred_element_type=f32)
                        + b2_r[...], 0.0)
        x = jnp.maximum(jnp.dot(x, w3_r[...], preferred_element_type=f32)
                        + b3_r[...], 0.0)
        g = ug_r[...] * ig_r[...]
        zg = jnp.sum(g * wfg_r[...], axis=1)
        zm = jnp.sum(x * wfm_r[...], axis=1)
        z = zg + zm + c_r[0, 0]
        out_r[...] = c_r[0, 1] / (1.0 + jnp.exp(-z)) + c_r[0, 2]

    full = lambda shape: pl.BlockSpec(shape, lambda i: (0, 0))
    row = lambda: pl.BlockSpec((BB, EMB), lambda i: (i, 0))
    return pl.pallas_call(
        body,
        grid=(grid,),
        in_specs=[
            row(), row(), row(), row(),
            full((1, EMB)),
            full((EMB, 128)), full((EMB, 128)), full((1, 128)),
            full((128, 128)), full((1, 128)),
            full((128, 128)), full((1, 128)),
            full((128, 128)), full((1, 128)),
            full((1, 128)), full((1, 128)),
        ],
        out_specs=pl.BlockSpec((BB,), lambda i: (i,)),
        out_shape=jax.ShapeDtypeStruct((BATCH,), jnp.float32),
    )(ug, ig, um, im, wfg, w0a, w0b, b0r, w1, b1r, w2, b2r, w3, b3r,
      wfm, consts)


def _pad2(a, r, c):
    return jnp.pad(a, ((0, r - a.shape[0]), (0, c - a.shape[1])))


def kernel(user, item, ue_gmf, ie_gmf, ue_mlp, ie_mlp,
           W0, b0, g0, beta0, W1, b1, g1, beta1,
           W2, b2, g2, beta2, W3, b3, g3, beta3,
           Wf, bf, scale, shift):
    ug, ig, um, im = _sc_gather4(user.astype(jnp.int32),
                                 item.astype(jnp.int32),
                                 ue_gmf, ie_gmf, ue_mlp, ie_mlp)

    inv = 1.0 / jnp.sqrt(jnp.float32(1.0 + BN_EPS))

    def fold(W, b, g, beta):
        s = inv * g
        return (W * s[:, None]).T, b * s + beta

    wt0, be0 = fold(W0, b0, g0, beta0)
    w0a, w0b = wt0[:EMB], wt0[EMB:]
    wt1, be1 = fold(W1, b1, g1, beta1)
    wt2, be2 = fold(W2, b2, g2, beta2)
    wt3, be3 = fold(W3, b3, g3, beta3)
    w1 = _pad2(wt1, 128, 128)
    w2 = _pad2(wt2, 128, 128)
    w3 = _pad2(wt3, 128, 128)
    b0r = be0.reshape(1, 128)
    b1r = _pad2(be1.reshape(1, -1), 1, 128)
    b2r = _pad2(be2.reshape(1, -1), 1, 128)
    b3r = _pad2(be3.reshape(1, -1), 1, 128)
    wfg = Wf[:, :EMB]
    wfm = _pad2(Wf[:, EMB:], 1, 128)
    consts = jnp.zeros((1, 128), jnp.float32)
    consts = consts.at[0, 0].set(bf[0]).at[0, 1].set(scale).at[0, 2].set(shift)

    return _tc_dense(ug, ig, um, im, wfg, w0a, w0b, b0r, w1, b1r,
                     w2, b2r, w3, b3r, wfm, consts)


# speedup vs baseline: 1.5101x; 1.5101x over previous
import functools

import jax
import jax.numpy as jnp
from jax import lax
from jax.experimental import pallas as pl
from jax.experimental.pallas import tpu as pltpu
from jax.experimental.pallas import tpu_sc as plsc

BATCH = 16384
EMB = 64
BN_EPS = 1e-5

NC = 2
NS = 16
NW = NC * NS
B_PER_W = BATCH // NW
CHUNK = 32
NCHUNK = B_PER_W // CHUNK
BLK = 8
NB = (NUM_ROWS := 1000000) // BLK


def _sc_gather4(user, item, t_ug, t_ig, t_um, t_im):
    mesh = plsc.VectorSubcoreMesh(core_axis_name="c", subcore_axis_name="s")

    out_t = [jax.ShapeDtypeStruct((BATCH, EMB), jnp.float32)] * 3
    scratch = [
        pltpu.VMEM((B_PER_W,), jnp.int32),
        pltpu.VMEM((B_PER_W,), jnp.int32),
    ] + [pltpu.VMEM((CHUNK, EMB), jnp.float32)] * 6 + [
        pltpu.SemaphoreType.DMA,
        pltpu.SemaphoreType.DMA,
    ]

    @functools.partial(pl.kernel, mesh=mesh, out_type=out_t,
                       scratch_types=scratch)
    def k(u_hbm, i_hbm, tug, tig, tum, tim,
          o_gmf, o_um, o_im,
          idx_u, idx_i, bu0, bu1, bi0, bi1, p0, p1, sem0, sem1):
        wid = lax.axis_index("s") * NC + lax.axis_index("c")
        base = wid * B_PER_W
        pltpu.sync_copy(u_hbm.at[pl.ds(base, B_PER_W)], idx_u)
        pltpu.sync_copy(i_hbm.at[pl.ds(base, B_PER_W)], idx_i)

        bu = (bu0, bu1)
        bi = (bi0, bi1)
        pb = (p0, p1)
        sems = (sem0, sem1)

        def fire(tu, ti, c, slot):
            for g in range(CHUNK // 16):
                uu = idx_u[pl.ds(c * CHUNK + g * 16, 16)]
                vv = idx_i[pl.ds(c * CHUNK + g * 16, 16)]
                for l in range(16):
                    j = g * 16 + l
                    pltpu.async_copy(tu.at[uu[l] >> 3, uu[l] & 7],
                                     bu[slot].at[j], sems[slot])
                    pltpu.async_copy(ti.at[vv[l] >> 3, vv[l] & 7],
                                     bi[slot].at[j], sems[slot])

        def drain(tu, slot):
            for _ in range(2 * CHUNK):
                pltpu.make_async_copy(tu.at[0, 0], bu[slot].at[0],
                                      sems[slot]).wait()

        def make_pass(tu, ti, do_prod, wb):
            def body2(t, carry):
                for k2 in range(2):
                    c = 2 * t + k2
                    slot = k2

                    @pl.when(c + 1 < NCHUNK)
                    def _():
                        fire(tu, ti, c + 1, 1 - k2)

                    drain(tu, slot)
                    if do_prod:
                        for l in range(CHUNK):
                            for q in range(EMB // 16):
                                cs = pl.ds(q * 16, 16)
                                pb[slot][l, cs] = (bu[slot][l, cs] *
                                                   bi[slot][l, cs])
                    wb(c, slot)
                return carry

            fire(tu, ti, 0, 0)
            lax.fori_loop(0, NCHUNK // 2, body2, 0)

        def wb_gmf(c, slot):
            pltpu.sync_copy(pb[slot],
                            o_gmf.at[pl.ds(base + c * CHUNK, CHUNK)])

        def wb_mlp(c, slot):
            pltpu.sync_copy(bu[slot],
                            o_um.at[pl.ds(base + c * CHUNK, CHUNK)])
            pltpu.sync_copy(bi[slot],
                            o_im.at[pl.ds(base + c * CHUNK, CHUNK)])

        make_pass(tug, tig, True, wb_gmf)
        make_pass(tum, tim, False, wb_mlp)

    tug3 = t_ug.reshape(NB, BLK, EMB)
    tig3 = t_ig.reshape(NB, BLK, EMB)
    tum3 = t_um.reshape(NB, BLK, EMB)
    tim3 = t_im.reshape(NB, BLK, EMB)
    return k(user, item, tug3, tig3, tum3, tim3)


def _tc_dense(gmf, um, im, wfg, w0a, w0b, b0r, w1, b1r, w2, b2r,
              w3, b3r, wfm, consts):
    BB = 2048
    grid = BATCH // BB

    def body(gmf_r, um_r, im_r, wfg_r, w0a_r, w0b_r, b0_r, w1_r, b1_r,
             w2_r, b2_r, w3_r, b3_r, wfm_r, c_r, out_r):
        f32 = jnp.float32
        x = jnp.dot(um_r[...], w0a_r[...], preferred_element_type=f32)
        x = x + jnp.dot(im_r[...], w0b_r[...], preferred_element_type=f32)
        x = jnp.maximum(x + b0_r[...], 0.0)
        x = jnp.maximum(jnp.dot(x, w1_r[...], preferred_element_type=f32)
                        + b1_r[...], 0.0)
        x = jnp.maximum(jnp.dot(x, w2_r[...], preferred_element_type=f32)
                        + b2_r[...], 0.0)
        x = jnp.maximum(jnp.dot(x, w3_r[...], preferred_element_type=f32)
                        + b3_r[...], 0.0)
        zg = jnp.sum(gmf_r[...] * wfg_r[...], axis=1)
        zm = jnp.sum(x * wfm_r[...], axis=1)
        z = zg + zm + c_r[0, 0]
        out_r[...] = c_r[0, 1] / (1.0 + jnp.exp(-z)) + c_r[0, 2]

    full = lambda shape: pl.BlockSpec(shape, lambda i: (0, 0))
    row = lambda: pl.BlockSpec((BB, EMB), lambda i: (i, 0))
    return pl.pallas_call(
        body,
        grid=(grid,),
        in_specs=[
            row(), row(), row(),
            full((1, EMB)),
            full((EMB, 128)), full((EMB, 128)), full((1, 128)),
            full((128, 128)), full((1, 128)),
            full((128, 128)), full((1, 128)),
            full((128, 128)), full((1, 128)),
            full((1, 128)), full((1, 128)),
        ],
        out_specs=pl.BlockSpec((BB,), lambda i: (i,)),
        out_shape=jax.ShapeDtypeStruct((BATCH,), jnp.float32),
    )(gmf, um, im, wfg, w0a, w0b, b0r, w1, b1r, w2, b2r, w3, b3r,
      wfm, consts)


def _pad2(a, r, c):
    return jnp.pad(a, ((0, r - a.shape[0]), (0, c - a.shape[1])))


def kernel(user, item, ue_gmf, ie_gmf, ue_mlp, ie_mlp,
           W0, b0, g0, beta0, W1, b1, g1, beta1,
           W2, b2, g2, beta2, W3, b3, g3, beta3,
           Wf, bf, scale, shift):
    gmf, um, im = _sc_gather4(user.astype(jnp.int32),
                              item.astype(jnp.int32),
                              ue_gmf, ie_gmf, ue_mlp, ie_mlp)

    inv = 1.0 / jnp.sqrt(jnp.float32(1.0 + BN_EPS))

    def fold(W, b, g, beta):
        s = inv * g
        return (W * s[:, None]).T, b * s + beta

    wt0, be0 = fold(W0, b0, g0, beta0)
    w0a, w0b = wt0[:EMB], wt0[EMB:]
    wt1, be1 = fold(W1, b1, g1, beta1)
    wt2, be2 = fold(W2, b2, g2, beta2)
    wt3, be3 = fold(W3, b3, g3, beta3)
    w1 = _pad2(wt1, 128, 128)
    w2 = _pad2(wt2, 128, 128)
    w3 = _pad2(wt3, 128, 128)
    b0r = be0.reshape(1, 128)
    b1r = _pad2(be1.reshape(1, -1), 1, 128)
    b2r = _pad2(be2.reshape(1, -1), 1, 128)
    b3r = _pad2(be3.reshape(1, -1), 1, 128)
    wfg = Wf[:, :EMB]
    wfm = _pad2(Wf[:, EMB:], 1, 128)
    consts = jnp.zeros((1, 128), jnp.float32)
    consts = consts.at[0, 0].set(bf[0]).at[0, 1].set(scale).at[0, 2].set(shift)

    return _tc_dense(gmf, um, im, wfg, w0a, w0b, b0r, w1, b1r,
                     w2, b2r, w3, b3r, wfm, consts)
